# baseline (speedup 1.0000x reference)
import functools

import jax
import jax.numpy as jnp
from jax import lax
from jax.experimental import pallas as pl
from jax.experimental.pallas import tpu as pltpu
from jax.experimental.pallas import tpu_sc as plsc

_NC = 2
_NS = 16
_NW = _NC * _NS
_CHUNK = 128
_NBUF = 5
_LOOKAHEAD = 3
_NCHUNKS = 5


@functools.lru_cache(maxsize=None)
def _make_xt(batch, t_steps):
    rows = (t_steps + 7) // 8 * 8

    def tr_kernel(x_ref, o_ref):
        o_ref[0:t_steps, :] = x_ref[...].T

    return pl.pallas_call(
        tr_kernel,
        in_specs=[pl.BlockSpec((batch, t_steps), lambda: (0, 0))],
        out_specs=pl.BlockSpec((rows, batch), lambda: (0, 0)),
        out_shape=jax.ShapeDtypeStruct((rows, batch), jnp.int32),
    )


@functools.lru_cache(maxsize=None)
def _make_sc_gather(vocab, emb, t0, t_chunk, t_steps, batch):
    assert batch == _NW * _CHUNK
    assert t_chunk % _NBUF == 0 and t_chunk % 2 == 0
    n_groups = t_chunk // _NBUF
    mesh = plsc.VectorSubcoreMesh(core_axis_name="c", subcore_axis_name="s")

    @functools.partial(
        pl.kernel,
        mesh=mesh,
        out_type=jax.ShapeDtypeStruct((t_chunk // 2, batch, 2 * emb), jnp.float32),
        scratch_types=[
            pltpu.VMEM((t_chunk, _CHUNK), jnp.int32),
            pltpu.VMEM((_NBUF, _CHUNK, emb), jnp.float32),
        ]
        + [pltpu.SemaphoreType.DMA] * (2 * _NBUF),
        compiler_params=pltpu.CompilerParams(
            use_tc_tiling_on_sc=False, needs_layout_passes=False
        ),
    )
    def gather_kernel(table_hbm, xt_hbm, out_hbm, idx_v, bufs, *sems):
        sem_g = sems[:_NBUF]
        sem_c = sems[_NBUF:]
        wid = lax.axis_index("s") * _NC + lax.axis_index("c")
        b0 = wid * _CHUNK

        pltpu.sync_copy(
            xt_hbm.at[pl.ds(t0, t_chunk), pl.ds(b0, _CHUNK)], idx_v
        )

        def out_slice(t):
            return out_hbm.at[t // 2, pl.ds(b0, _CHUNK), pl.ds((t % 2) * emb, emb)]

        def gather_issue(t, b):
            pltpu.async_copy(table_hbm.at[idx_v.at[t]], bufs.at[b], sem_g[b])

        def gather_wait(t, b):
            pltpu.make_async_copy(
                table_hbm.at[idx_v.at[t]], bufs.at[b], sem_g[b]
            ).wait()

        def copyout_issue(t, b):
            pltpu.async_copy(bufs.at[b], out_slice(t), sem_c[b])

        def copyout_wait(t, b):
            pltpu.make_async_copy(bufs.at[b], out_slice(t), sem_c[b]).wait()

        for b in range(_LOOKAHEAD):
            gather_issue(b, b)

        def group(g, carry):
            for b in range(_NBUF):
                t = g * _NBUF + b
                gather_wait(t, b)
                copyout_issue(t, b)
                k = t + _LOOKAHEAD
                nb = (b + _LOOKAHEAD) % _NBUF

                @pl.when(k < t_chunk)
                def _():
                    @pl.when(k >= _NBUF)
                    def _():
                        copyout_wait(k - _NBUF, nb)

                    gather_issue(k, nb)

            return carry

        lax.fori_loop(0, n_groups, group, 0)

        for b in range(_NBUF):
            copyout_wait(t_chunk - _NBUF + b, b)

    return gather_kernel


def _rnn_steps(emb_ref, wih_ref, whh_ref, b_ref, h, emb):
    x2 = emb_ref[0].astype(jnp.bfloat16)
    z_e = jax.lax.dot_general(
        x2[:, 0:emb], wih_ref[...], (((1,), (0,)), ((), ())),
        preferred_element_type=jnp.float32,
    )
    z_o = jax.lax.dot_general(
        x2[:, emb : 2 * emb], wih_ref[...], (((1,), (0,)), ((), ())),
        preferred_element_type=jnp.float32,
    )
    h = jnp.tanh(
        z_e
        + jnp.dot(h, whh_ref[...], preferred_element_type=jnp.float32)
        + b_ref[...]
    )
    h = jnp.tanh(
        z_o
        + jnp.dot(h, whh_ref[...], preferred_element_type=jnp.float32)
        + b_ref[...]
    )
    return h


def _rnn_in_specs(batch, emb, hid, with_h):
    specs = [pl.BlockSpec((1, batch, 2 * emb), lambda u: (u, 0, 0))]
    if with_h:
        specs.append(pl.BlockSpec((batch, hid), lambda u: (0, 0)))
    return specs + [
        pl.BlockSpec((emb, hid), lambda u: (0, 0)),
        pl.BlockSpec((hid, hid), lambda u: (0, 0)),
        pl.BlockSpec((1, hid), lambda u: (0, 0)),
    ]


@functools.lru_cache(maxsize=None)
def _make_rnn_chunk(t_chunk, batch, emb, hid, first):
    def body(emb_ref, hin_ref, wih_ref, whh_ref, b_ref, hout_ref):
        u = pl.program_id(0)

        @pl.when(u == 0)
        def _():
            hout_ref[...] = (
                jnp.zeros_like(hout_ref) if hin_ref is None else hin_ref[...]
            )

        hout_ref[...] = _rnn_steps(
            emb_ref, wih_ref, whh_ref, b_ref, hout_ref[...], emb
        )

    if first:
        def rnn_kernel(emb_ref, wih_ref, whh_ref, b_ref, hout_ref):
            body(emb_ref, None, wih_ref, whh_ref, b_ref, hout_ref)
    else:
        rnn_kernel = body

    return pl.pallas_call(
        rnn_kernel,
        grid=(t_chunk // 2,),
        in_specs=_rnn_in_specs(batch, emb, hid, with_h=not first),
        out_specs=pl.BlockSpec((batch, hid), lambda u: (0, 0)),
        out_shape=jax.ShapeDtypeStruct((batch, hid), jnp.float32),
        compiler_params=pltpu.CompilerParams(
            dimension_semantics=("arbitrary",),
        ),
    )


@functools.lru_cache(maxsize=None)
def _make_rnn_final(t_chunk, batch, emb, hid, out_dim):
    n_pairs = t_chunk // 2

    def rnn_kernel(emb_ref, hin_ref, wih_ref, whh_ref, b_ref,
                   wfc_ref, bfc_ref, out_ref, h_ref):
        u = pl.program_id(0)

        @pl.when(u == 0)
        def _():
            h_ref[...] = hin_ref[...]

        h = _rnn_steps(emb_ref, wih_ref, whh_ref, b_ref, h_ref[...], emb)
        h_ref[...] = h

        @pl.when(u == n_pairs - 1)
        def _():
            out_ref[...] = (
                jnp.dot(h, wfc_ref[...], preferred_element_type=jnp.float32)
                + bfc_ref[...]
            )

    return pl.pallas_call(
        rnn_kernel,
        grid=(n_pairs,),
        in_specs=_rnn_in_specs(batch, emb, hid, with_h=True)
        + [
            pl.BlockSpec((hid, out_dim), lambda u: (0, 0)),
            pl.BlockSpec((1, out_dim), lambda u: (0, 0)),
        ],
        out_specs=pl.BlockSpec((batch, out_dim), lambda u: (0, 0)),
        out_shape=jax.ShapeDtypeStruct((batch, out_dim), jnp.float32),
        scratch_shapes=[pltpu.VMEM((batch, hid), jnp.float32)],
        compiler_params=pltpu.CompilerParams(
            dimension_semantics=("arbitrary",),
        ),
    )


def kernel(x, embeddings, W_ih, W_hh, b_ih, b_hh, W_fc, b_fc):
    batch, t_steps = x.shape
    vocab, emb = embeddings.shape
    hid = W_ih.shape[0]
    out_dim = W_fc.shape[0]
    lens = []
    rem = t_steps
    first = 0
    if 0 < first < t_steps and (t_steps - first) % (2 * _NBUF) == 0:
        lens = [first, t_steps - first]
    else:
        lens = [t_steps]
    starts = [sum(lens[:i]) for i in range(len(lens))]

    xt = _make_xt(batch, t_steps)(x)

    embs = [
        _make_sc_gather(vocab, emb, t0, tc, t_steps, batch)(embeddings, xt)
        for t0, tc in zip(starts, lens)
    ]

    b2 = (b_ih + b_hh).reshape(1, hid)
    wih_bf = W_ih.T.astype(jnp.bfloat16)

    h = None
    for c in range(len(lens) - 1):
        args = (embs[c],) if h is None else (embs[c], h)
        h = _make_rnn_chunk(lens[c], batch, emb, hid, first=(h is None))(
            *args, wih_bf, W_hh.T, b2
        )
    if h is None:
        h = jnp.zeros((batch, hid), jnp.float32)
    logits = _make_rnn_final(lens[-1], batch, emb, hid, out_dim)(
        embs[-1], h, wih_bf, W_hh.T, b2, W_fc.T, b_fc.reshape(1, out_dim)
    )
    return logits


# speedup vs baseline: 1.0377x; 1.0377x over previous
import functools

import jax
import jax.numpy as jnp
from jax import lax
from jax.experimental import pallas as pl
from jax.experimental.pallas import tpu as pltpu
from jax.experimental.pallas import tpu_sc as plsc

_NC = 2
_NS = 16
_NW = _NC * _NS
_CHUNK = 128
_NBUF = 5
_LOOKAHEAD = 3


@functools.lru_cache(maxsize=None)
def _make_xt(batch, t_steps):
    rows = (t_steps + 7) // 8 * 8

    def tr_kernel(x_ref, o_ref):
        o_ref[0:t_steps, :] = x_ref[...].T

    return pl.pallas_call(
        tr_kernel,
        in_specs=[pl.BlockSpec((batch, t_steps), lambda: (0, 0))],
        out_specs=pl.BlockSpec((rows, batch), lambda: (0, 0)),
        out_shape=jax.ShapeDtypeStruct((rows, batch), jnp.int32),
    )


@functools.lru_cache(maxsize=None)
def _make_sc_gather(vocab, emb, t0, t_chunk, t_steps, batch):
    assert batch == _NW * _CHUNK
    assert t_chunk % _NBUF == 0 and t_chunk % 2 == 0
    n_groups = t_chunk // _NBUF
    mesh = plsc.VectorSubcoreMesh(core_axis_name="c", subcore_axis_name="s")

    @functools.partial(
        pl.kernel,
        mesh=mesh,
        out_type=jax.ShapeDtypeStruct((t_chunk // 2, batch, 2 * emb), jnp.float32),
        scratch_types=[
            pltpu.VMEM((t_chunk, _CHUNK), jnp.int32),
            pltpu.VMEM((_NBUF, _CHUNK, emb), jnp.float32),
        ]
        + [pltpu.SemaphoreType.DMA] * (2 * _NBUF),
        compiler_params=pltpu.CompilerParams(
            use_tc_tiling_on_sc=False, needs_layout_passes=False
        ),
    )
    def gather_kernel(table_hbm, xt_hbm, out_hbm, idx_v, bufs, *sems):
        sem_g = sems[:_NBUF]
        sem_c = sems[_NBUF:]
        wid = lax.axis_index("s") * _NC + lax.axis_index("c")
        b0 = wid * _CHUNK

        pltpu.sync_copy(
            xt_hbm.at[pl.ds(t0, t_chunk), pl.ds(b0, _CHUNK)], idx_v
        )

        def out_slice(t):
            return out_hbm.at[t // 2, pl.ds(b0, _CHUNK), pl.ds((t % 2) * emb, emb)]

        def gather_issue(t, b):
            pltpu.async_copy(table_hbm.at[idx_v.at[t]], bufs.at[b], sem_g[b])

        def gather_wait(t, b):
            pltpu.make_async_copy(
                table_hbm.at[idx_v.at[t]], bufs.at[b], sem_g[b]
            ).wait()

        def copyout_issue(t, b):
            pltpu.async_copy(bufs.at[b], out_slice(t), sem_c[b])

        def copyout_wait(t, b):
            pltpu.make_async_copy(bufs.at[b], out_slice(t), sem_c[b]).wait()

        for b in range(_LOOKAHEAD):
            gather_issue(b, b)

        def group(g, carry):
            for b in range(_NBUF):
                t = g * _NBUF + b
                gather_wait(t, b)
                copyout_issue(t, b)
                k = t + _LOOKAHEAD
                nb = (b + _LOOKAHEAD) % _NBUF

                @pl.when(k < t_chunk)
                def _():
                    @pl.when(k >= _NBUF)
                    def _():
                        copyout_wait(k - _NBUF, nb)

                    gather_issue(k, nb)

            return carry

        lax.fori_loop(0, n_groups, group, 0)

        for b in range(_NBUF):
            copyout_wait(t_chunk - _NBUF + b, b)

    return gather_kernel


def _rnn_steps(emb_ref, wih_ref, whh_ref, b_ref, h, emb):
    x2 = emb_ref[0].astype(jnp.bfloat16)
    z_e = jax.lax.dot_general(
        x2[:, 0:emb], wih_ref[...], (((1,), (0,)), ((), ())),
        preferred_element_type=jnp.float32,
    )
    z_o = jax.lax.dot_general(
        x2[:, emb : 2 * emb], wih_ref[...], (((1,), (0,)), ((), ())),
        preferred_element_type=jnp.float32,
    )
    h = jnp.tanh(
        z_e
        + jnp.dot(h, whh_ref[...], preferred_element_type=jnp.float32)
        + b_ref[...]
    )
    h = jnp.tanh(
        z_o
        + jnp.dot(h, whh_ref[...], preferred_element_type=jnp.float32)
        + b_ref[...]
    )
    return h


def _rnn_in_specs(batch, emb, hid, with_h):
    specs = [pl.BlockSpec((1, batch, 2 * emb), lambda u: (u, 0, 0))]
    if with_h:
        specs.append(pl.BlockSpec((batch, hid), lambda u: (0, 0)))
    return specs + [
        pl.BlockSpec((emb, hid), lambda u: (0, 0)),
        pl.BlockSpec((hid, hid), lambda u: (0, 0)),
        pl.BlockSpec((1, hid), lambda u: (0, 0)),
    ]


@functools.lru_cache(maxsize=None)
def _make_rnn_chunk(t_chunk, batch, emb, hid, first):
    def body(emb_ref, hin_ref, wih_ref, whh_ref, b_ref, hout_ref):
        u = pl.program_id(0)

        @pl.when(u == 0)
        def _():
            hout_ref[...] = (
                jnp.zeros_like(hout_ref) if hin_ref is None else hin_ref[...]
            )

        hout_ref[...] = _rnn_steps(
            emb_ref, wih_ref, whh_ref, b_ref, hout_ref[...], emb
        )

    if first:
        def rnn_kernel(emb_ref, wih_ref, whh_ref, b_ref, hout_ref):
            body(emb_ref, None, wih_ref, whh_ref, b_ref, hout_ref)
    else:
        rnn_kernel = body

    return pl.pallas_call(
        rnn_kernel,
        grid=(t_chunk // 2,),
        in_specs=_rnn_in_specs(batch, emb, hid, with_h=not first),
        out_specs=pl.BlockSpec((batch, hid), lambda u: (0, 0)),
        out_shape=jax.ShapeDtypeStruct((batch, hid), jnp.float32),
        compiler_params=pltpu.CompilerParams(
            dimension_semantics=("arbitrary",),
        ),
    )


@functools.lru_cache(maxsize=None)
def _make_rnn_final(t_chunk, batch, emb, hid, out_dim):
    n_pairs = t_chunk // 2

    def rnn_kernel(emb_ref, hin_ref, wih_ref, whh_ref, b_ref,
                   wfc_ref, bfc_ref, out_ref, h_ref):
        u = pl.program_id(0)

        @pl.when(u == 0)
        def _():
            h_ref[...] = hin_ref[...]

        h = _rnn_steps(emb_ref, wih_ref, whh_ref, b_ref, h_ref[...], emb)
        h_ref[...] = h

        @pl.when(u == n_pairs - 1)
        def _():
            out_ref[...] = (
                jnp.dot(h, wfc_ref[...], preferred_element_type=jnp.float32)
                + bfc_ref[...]
            )

    return pl.pallas_call(
        rnn_kernel,
        grid=(n_pairs,),
        in_specs=_rnn_in_specs(batch, emb, hid, with_h=True)
        + [
            pl.BlockSpec((hid, out_dim), lambda u: (0, 0)),
            pl.BlockSpec((1, out_dim), lambda u: (0, 0)),
        ],
        out_specs=pl.BlockSpec((batch, out_dim), lambda u: (0, 0)),
        out_shape=jax.ShapeDtypeStruct((batch, out_dim), jnp.float32),
        scratch_shapes=[pltpu.VMEM((batch, hid), jnp.float32)],
        compiler_params=pltpu.CompilerParams(
            dimension_semantics=("arbitrary",),
        ),
    )


def kernel(x, embeddings, W_ih, W_hh, b_ih, b_hh, W_fc, b_fc):
    batch, t_steps = x.shape
    vocab, emb = embeddings.shape
    hid = W_ih.shape[0]
    out_dim = W_fc.shape[0]
    first = (2 * t_steps // 5) // (2 * _NBUF) * (2 * _NBUF)
    if 0 < first < t_steps and (t_steps - first) % (2 * _NBUF) == 0:
        lens = [first, t_steps - first]
    else:
        lens = [t_steps]
    starts = [sum(lens[:i]) for i in range(len(lens))]

    xt = _make_xt(batch, t_steps)(x)

    embs = [
        _make_sc_gather(vocab, emb, t0, tc, t_steps, batch)(embeddings, xt)
        for t0, tc in zip(starts, lens)
    ]

    b2 = (b_ih + b_hh).reshape(1, hid)
    wih_bf = W_ih.T.astype(jnp.bfloat16)

    h = None
    for c in range(len(lens) - 1):
        args = (embs[c],) if h is None else (embs[c], h)
        h = _make_rnn_chunk(lens[c], batch, emb, hid, first=(h is None))(
            *args, wih_bf, W_hh.T, b2
        )
    if h is None:
        h = jnp.zeros((batch, hid), jnp.float32)
    logits = _make_rnn_final(lens[-1], batch, emb, hid, out_dim)(
        embs[-1], h, wih_bf, W_hh.T, b2, W_fc.T, b_fc.reshape(1, out_dim)
    )
    return logits
